# baseline (speedup 1.0000x reference)
import jax
import jax.numpy as jnp
from jax import lax
from jax.experimental import pallas as pl
from jax.experimental.pallas import tpu as pltpu
from jax.experimental.pallas import tpu_sc as plsc

_B, _L, _D = 4096, 200, 32
_N = _B * _L
_W = 512
_CB = _B // _W


def _gather_sc(table, idx):
    mesh = plsc.VectorSubcoreMesh(core_axis_name="c", subcore_axis_name="s")

    @pl.kernel(
        out_type=jax.ShapeDtypeStruct((_N, _D), jnp.float32),
        mesh=mesh,
        compiler_params=pltpu.CompilerParams(use_tc_tiling_on_sc=False),
    )
    def _embed(table_hbm, idx_hbm, out_hbm):
        def body(i_vmem, o_vmem):
            pltpu.sync_copy(table_hbm.at[i_vmem], o_vmem)

        pltpu.emit_pipeline(
            body,
            grid=(_N // _W,),
            in_specs=[pl.BlockSpec((_W,), index_map=lambda i: (i,))],
            out_specs=[pl.BlockSpec((_W, _D), index_map=lambda i: (i, 0))],
            core_axis_name=("c", "s"),
            dimension_semantics=(pltpu.PARALLEL,),
        )(idx_hbm, out_hbm)

    return _embed(table, idx)


def _transpose_tc(g2):
    def body(x_ref, o_ref):
        c = pl.program_id(1)
        t = x_ref[...].T
        b_iota = lax.broadcasted_iota(jnp.int32, (_D, _W), 1)
        first = jnp.logical_and(c == 0, b_iota == 0)
        o_ref[0] = jnp.where(first, jnp.float32(0), t)

    return pl.pallas_call(
        body,
        grid=(_L, _CB),
        in_specs=[pl.BlockSpec((_W, _D), lambda l, c: (l * _CB + c, 0))],
        out_specs=pl.BlockSpec((1, _D, _W), lambda l, c: (l, 0, c)),
        out_shape=jax.ShapeDtypeStruct((_L, _D, _B), jnp.float32),
    )(g2)


def kernel(y, table):
    yt = y.T.astype(jnp.int32)
    idx2 = jnp.pad(yt[:, :-1], ((0, 0), (1, 0)))
    idx2 = idx2.reshape(_N)
    g2 = _gather_sc(table, idx2)
    p = _transpose_tc(g2)
    return jnp.transpose(p, (2, 0, 1))
